# baseline (speedup 1.0000x reference)
import functools

import jax
import jax.numpy as jnp
from jax import lax
from jax.experimental import pallas as pl
from jax.experimental.pallas import tpu as pltpu
from jax.experimental.pallas import tpu_sc as plsc

N = 10000
E = 320000
D = 128
H = 128
C = 10
NG = 64

NC = 2
NS = 16
NW = NC * NS
EW = E // NW
K = 80
J = EW // K
JP = 64
KD = 80
JD = EW // KD
SB = 640
SBL = N - SB * (NS - 1)
DW = 8

_mesh = plsc.VectorSubcoreMesh(
    core_axis_name="c", subcore_axis_name="s", num_cores=NC, num_subcores=NS)


@functools.partial(
    pl.kernel,
    out_type=jax.ShapeDtypeStruct((NC, N, DW), jnp.float32),
    mesh=_mesh,
    scratch_types=[
        pltpu.VMEM((JD, KD), jnp.int32),
        pltpu.VMEM((KD, DW), jnp.float32),
        pltpu.VMEM_SHARED((N, DW), jnp.float32),
    ],
)
def _deg_kernel(edge_hbm, zeros_hbm, degp_hbm, dst_v, ones_v, acc_sh):
    c = lax.axis_index("c")
    s = lax.axis_index("s")
    w = s * NC + c

    @pl.when(s < NS - 1)
    def _():
        pltpu.sync_copy(zeros_hbm, acc_sh.at[pl.ds(s * 640, 640)])

    @pl.when(s == NS - 1)
    def _():
        pltpu.sync_copy(zeros_hbm.at[pl.ds(0, 400)],
                        acc_sh.at[pl.ds((NS - 1) * 640, 400)])

    pltpu.sync_copy(edge_hbm.at[1, w], dst_v)
    for i in range(KD // 16):
        ones_v[pl.ds(i * 16, 16), :] = jnp.ones((16, DW), jnp.float32)
    plsc.subcore_barrier()

    def body(j, carry):
        pltpu.sync_copy(ones_v, acc_sh.at[dst_v.at[j]], add=True)
        return carry

    lax.fori_loop(0, JD, body, 0)
    plsc.subcore_barrier()

    @pl.when(s < NS - 1)
    def _():
        pltpu.sync_copy(acc_sh.at[pl.ds(s * 640, 640)],
                        degp_hbm.at[c, pl.ds(s * 640, 640)])

    @pl.when(s == NS - 1)
    def _():
        pltpu.sync_copy(acc_sh.at[pl.ds((NS - 1) * 640, 400)],
                        degp_hbm.at[c, pl.ds((NS - 1) * 640, 400)])


@functools.partial(
    pl.kernel,
    out_type=jax.ShapeDtypeStruct((NC, N, H), jnp.float32),
    mesh=_mesh,
    scratch_types=[
        pltpu.VMEM((JP, K), jnp.int32),
        pltpu.VMEM((JP, K), jnp.int32),
        pltpu.VMEM((K, H), jnp.float32),
        pltpu.VMEM((K, H), jnp.float32),
        pltpu.VMEM_SHARED((N, H), jnp.float32),
        pltpu.SemaphoreType.DMA,
        pltpu.SemaphoreType.DMA,
        pltpu.SemaphoreType.DMA,
        pltpu.SemaphoreType.DMA,
    ],
)
def _msg_kernel(hs_hbm, edge_hbm, zeros_hbm, acc_hbm,
                src_v, dst_v, rows0_v, rows1_v, acc_sh,
                sem0, sem1, ssem0, ssem1):
    c = lax.axis_index("c")
    s = lax.axis_index("s")
    w = s * NC + c

    @pl.when(s < NS - 1)
    def _():
        pltpu.sync_copy(zeros_hbm, acc_sh.at[pl.ds(s * SB, SB)])

    @pl.when(s == NS - 1)
    def _():
        pltpu.sync_copy(zeros_hbm.at[pl.ds(0, SBL)],
                        acc_sh.at[pl.ds((NS - 1) * SB, SBL)])

    plsc.subcore_barrier()

    def wait_gather(buf, sem):
        pltpu.make_async_copy(hs_hbm.at[src_v.at[0]], buf, sem).wait()

    def wait_scatter(buf, sem):
        pltpu.make_async_copy(buf, acc_sh.at[dst_v.at[0]], sem).wait()

    def run_phase(base, count):
        pltpu.sync_copy(edge_hbm.at[0, w, pl.ds(base, count)],
                        src_v.at[pl.ds(0, count)])
        pltpu.sync_copy(edge_hbm.at[1, w, pl.ds(base, count)],
                        dst_v.at[pl.ds(0, count)])
        pltpu.async_copy(hs_hbm.at[src_v.at[0]], rows0_v, sem0)
        pltpu.async_copy(hs_hbm.at[src_v.at[1]], rows1_v, sem1)
        wait_gather(rows0_v, sem0)
        pltpu.async_copy(rows0_v, acc_sh.at[dst_v.at[0]], ssem0, add=True)

        def body(j, carry):
            even = (j % 2) == 0

            @pl.when(even)
            def _():
                wait_scatter(rows1_v, ssem1)
                pltpu.async_copy(hs_hbm.at[src_v.at[j + 1]], rows1_v, sem1)
                wait_gather(rows0_v, sem0)
                pltpu.async_copy(rows0_v, acc_sh.at[dst_v.at[j]], ssem0,
                                 add=True)

            @pl.when(jnp.logical_not(even))
            def _():
                wait_scatter(rows0_v, ssem0)
                pltpu.async_copy(hs_hbm.at[src_v.at[j + 1]], rows0_v, sem0)
                wait_gather(rows1_v, sem1)
                pltpu.async_copy(rows1_v, acc_sh.at[dst_v.at[j]], ssem1,
                                 add=True)

            return carry

        lax.fori_loop(1, count - 1, body, 0)
        if (count - 1) % 2:
            wait_gather(rows1_v, sem1)
            pltpu.async_copy(rows1_v, acc_sh.at[dst_v.at[count - 1]], ssem1,
                             add=True)
            wait_scatter(rows0_v, ssem0)
            wait_scatter(rows1_v, ssem1)
        else:
            wait_gather(rows0_v, sem0)
            pltpu.async_copy(rows0_v, acc_sh.at[dst_v.at[count - 1]], ssem0,
                             add=True)
            wait_scatter(rows1_v, ssem1)
            wait_scatter(rows0_v, ssem0)

    run_phase(0, JP)
    run_phase(JP, J - JP)
    plsc.subcore_barrier()

    @pl.when(s < NS - 1)
    def _():
        pltpu.sync_copy(acc_sh.at[pl.ds(s * SB, SB)],
                        acc_hbm.at[c, pl.ds(s * SB, SB)])

    @pl.when(s == NS - 1)
    def _():
        pltpu.sync_copy(acc_sh.at[pl.ds((NS - 1) * SB, SBL)],
                        acc_hbm.at[c, pl.ds((NS - 1) * SB, SBL)])


def _dis(degp_ref):
    return lax.rsqrt(degp_ref[0, :, 0:1] + degp_ref[1, :, 0:1] + 1.0)


def _dense_in_body(degp_ref, x_ref, w_ref, b_ref, hs_ref):
    hh = jnp.dot(x_ref[...], w_ref[...],
                 preferred_element_type=jnp.float32) + b_ref[...]
    hs_ref[...] = hh * _dis(degp_ref)


def _dense_mid_body(acc_ref, hs_ref, degp_ref, g_ref, be_ref, w_ref, b_ref,
                    out_ref):
    dis = _dis(degp_ref)
    z = (acc_ref[0] + acc_ref[1] + hs_ref[...]) * dis
    mu = jnp.mean(z, axis=0, keepdims=True)
    var = jnp.mean((z - mu) ** 2, axis=0, keepdims=True)
    h = (z - mu) * lax.rsqrt(var + 1e-5) * g_ref[...] + be_ref[...]
    h = jnp.maximum(h, 0.0)
    hh = jnp.dot(h, w_ref[...],
                 preferred_element_type=jnp.float32) + b_ref[...]
    out_ref[...] = hh * dis


def _dense_out_body(acc_ref, hs_ref, degp_ref, g_ref, be_ref, batch_ref,
                    wc1_ref, bc1_ref, wc2_ref, bc2_ref, out_ref):
    z = (acc_ref[0] + acc_ref[1] + hs_ref[...]) * _dis(degp_ref)
    mu = jnp.mean(z, axis=0, keepdims=True)
    var = jnp.mean((z - mu) ** 2, axis=0, keepdims=True)
    h = (z - mu) * lax.rsqrt(var + 1e-5) * g_ref[...] + be_ref[...]
    h = jnp.maximum(h, 0.0)
    gi = lax.broadcasted_iota(jnp.int32, (NG, N), 0)
    onehot = (gi == batch_ref[...]).astype(jnp.float32)
    pool = jnp.dot(onehot, h, preferred_element_type=jnp.float32)
    c1 = jnp.maximum(
        jnp.dot(pool, wc1_ref[...], preferred_element_type=jnp.float32)
        + bc1_ref[...], 0.0)
    out_ref[...] = jnp.dot(c1, wc2_ref[...],
                           preferred_element_type=jnp.float32) + bc2_ref[...]


def kernel(x, edge_index, batch, W1, b1, g1, be1, W2, b2, g2, be2,
           W3, b3, g3, be3, Wc1, bc1, Wc2, bc2):
    f32 = jnp.float32
    i32 = jnp.int32
    ei = edge_index.astype(i32)
    edge_r = ei.reshape(2, NW, J, K)
    edge_d = ei.reshape(2, NW, JD, KD)
    zeros_deg = jnp.zeros((640, DW), f32)
    zeros_rows = jnp.zeros((SB, H), f32)
    batch2d = batch.astype(i32).reshape(1, N)

    degp = _deg_kernel(edge_d, zeros_deg)

    dense_in = pl.pallas_call(
        _dense_in_body,
        out_shape=jax.ShapeDtypeStruct((N, H), f32))
    hs = dense_in(degp, x, W1, b1.reshape(1, H))

    dense_mid = pl.pallas_call(
        _dense_mid_body,
        out_shape=jax.ShapeDtypeStruct((N, H), f32))
    for (g, be, W, b) in ((g1, be1, W2, b2), (g2, be2, W3, b3)):
        acc = _msg_kernel(hs, edge_r, zeros_rows)
        hs = dense_mid(acc, hs, degp, g.reshape(1, H), be.reshape(1, H),
                       W, b.reshape(1, H))

    acc = _msg_kernel(hs, edge_r, zeros_rows)
    dense_out = pl.pallas_call(
        _dense_out_body,
        out_shape=jax.ShapeDtypeStruct((NG, C), f32))
    out = dense_out(acc, hs, degp, g3.reshape(1, H), be3.reshape(1, H),
                    batch2d, Wc1, bc1.reshape(1, H), Wc2, bc2.reshape(1, C))
    return out


# speedup vs baseline: 1.0087x; 1.0087x over previous
import functools

import jax
import jax.numpy as jnp
from jax import lax
from jax.experimental import pallas as pl
from jax.experimental.pallas import tpu as pltpu
from jax.experimental.pallas import tpu_sc as plsc

N = 10000
E = 320000
D = 128
H = 128
C = 10
NG = 64

NC = 2
NS = 16
NW = NC * NS
EW = E // NW
K = 80
J = EW // K
JP = 64
KD = 80
JD = EW // KD
SB = 640
SBL = N - SB * (NS - 1)
DW = 8

_mesh = plsc.VectorSubcoreMesh(
    core_axis_name="c", subcore_axis_name="s", num_cores=NC, num_subcores=NS)


@functools.partial(
    pl.kernel,
    out_type=jax.ShapeDtypeStruct((NC, N, DW), jnp.float32),
    mesh=_mesh,
    scratch_types=[
        pltpu.VMEM((JD, KD), jnp.int32),
        pltpu.VMEM((KD, DW), jnp.float32),
        pltpu.VMEM_SHARED((N, DW), jnp.float32),
    ],
)
def _deg_kernel(edge_hbm, zeros_hbm, degp_hbm, dst_v, ones_v, acc_sh):
    c = lax.axis_index("c")
    s = lax.axis_index("s")
    w = s * NC + c

    @pl.when(s < NS - 1)
    def _():
        pltpu.sync_copy(zeros_hbm, acc_sh.at[pl.ds(s * 640, 640)])

    @pl.when(s == NS - 1)
    def _():
        pltpu.sync_copy(zeros_hbm.at[pl.ds(0, 400)],
                        acc_sh.at[pl.ds((NS - 1) * 640, 400)])

    pltpu.sync_copy(edge_hbm.at[1, w], dst_v)
    for i in range(KD // 16):
        ones_v[pl.ds(i * 16, 16), :] = jnp.ones((16, DW), jnp.float32)
    plsc.subcore_barrier()

    def body(j, carry):
        pltpu.sync_copy(ones_v, acc_sh.at[dst_v.at[j]], add=True)
        return carry

    lax.fori_loop(0, JD, body, 0)
    plsc.subcore_barrier()

    @pl.when(s < NS - 1)
    def _():
        pltpu.sync_copy(acc_sh.at[pl.ds(s * 640, 640)],
                        degp_hbm.at[c, pl.ds(s * 640, 640)])

    @pl.when(s == NS - 1)
    def _():
        pltpu.sync_copy(acc_sh.at[pl.ds((NS - 1) * 640, 400)],
                        degp_hbm.at[c, pl.ds((NS - 1) * 640, 400)])


@functools.partial(
    pl.kernel,
    out_type=jax.ShapeDtypeStruct((NC, N, H), jnp.float32),
    mesh=_mesh,
    scratch_types=[
        pltpu.VMEM((JP, K), jnp.int32),
        pltpu.VMEM((JP, K), jnp.int32),
        pltpu.VMEM((K, H), jnp.float32),
        pltpu.VMEM((K, H), jnp.float32),
        pltpu.VMEM_SHARED((N, H), jnp.float32),
        pltpu.SemaphoreType.DMA,
        pltpu.SemaphoreType.DMA,
        pltpu.SemaphoreType.DMA,
        pltpu.SemaphoreType.DMA,
    ],
)
def _msg_kernel(hs_hbm, edge_hbm, zeros_hbm, acc_hbm,
                src_v, dst_v, rows0_v, rows1_v, acc_sh,
                sem0, sem1, ssem0, ssem1):
    c = lax.axis_index("c")
    s = lax.axis_index("s")
    w = s * NC + c

    @pl.when(jnp.logical_and(c == 0, s < NS - 1))
    def _():
        pltpu.sync_copy(hs_hbm.at[pl.ds(s * SB, SB)],
                        acc_sh.at[pl.ds(s * SB, SB)])

    @pl.when(jnp.logical_and(c == 0, s == NS - 1))
    def _():
        pltpu.sync_copy(hs_hbm.at[pl.ds((NS - 1) * SB, SBL)],
                        acc_sh.at[pl.ds((NS - 1) * SB, SBL)])

    @pl.when(jnp.logical_and(c == 1, s < NS - 1))
    def _():
        pltpu.sync_copy(zeros_hbm, acc_sh.at[pl.ds(s * SB, SB)])

    @pl.when(jnp.logical_and(c == 1, s == NS - 1))
    def _():
        pltpu.sync_copy(zeros_hbm.at[pl.ds(0, SBL)],
                        acc_sh.at[pl.ds((NS - 1) * SB, SBL)])

    plsc.subcore_barrier()

    def wait_gather(buf, sem):
        pltpu.make_async_copy(hs_hbm.at[src_v.at[0]], buf, sem).wait()

    def wait_scatter(buf, sem):
        pltpu.make_async_copy(buf, acc_sh.at[dst_v.at[0]], sem).wait()

    def run_phase(base, count):
        pltpu.sync_copy(edge_hbm.at[0, w, pl.ds(base, count)],
                        src_v.at[pl.ds(0, count)])
        pltpu.sync_copy(edge_hbm.at[1, w, pl.ds(base, count)],
                        dst_v.at[pl.ds(0, count)])
        pltpu.async_copy(hs_hbm.at[src_v.at[0]], rows0_v, sem0)
        pltpu.async_copy(hs_hbm.at[src_v.at[1]], rows1_v, sem1)
        wait_gather(rows0_v, sem0)
        pltpu.async_copy(rows0_v, acc_sh.at[dst_v.at[0]], ssem0, add=True)

        def body(j, carry):
            even = (j % 2) == 0

            @pl.when(even)
            def _():
                wait_scatter(rows1_v, ssem1)
                pltpu.async_copy(hs_hbm.at[src_v.at[j + 1]], rows1_v, sem1)
                wait_gather(rows0_v, sem0)
                pltpu.async_copy(rows0_v, acc_sh.at[dst_v.at[j]], ssem0,
                                 add=True)

            @pl.when(jnp.logical_not(even))
            def _():
                wait_scatter(rows0_v, ssem0)
                pltpu.async_copy(hs_hbm.at[src_v.at[j + 1]], rows0_v, sem0)
                wait_gather(rows1_v, sem1)
                pltpu.async_copy(rows1_v, acc_sh.at[dst_v.at[j]], ssem1,
                                 add=True)

            return carry

        lax.fori_loop(1, count - 1, body, 0)
        if (count - 1) % 2:
            wait_gather(rows1_v, sem1)
            pltpu.async_copy(rows1_v, acc_sh.at[dst_v.at[count - 1]], ssem1,
                             add=True)
            wait_scatter(rows0_v, ssem0)
            wait_scatter(rows1_v, ssem1)
        else:
            wait_gather(rows0_v, sem0)
            pltpu.async_copy(rows0_v, acc_sh.at[dst_v.at[count - 1]], ssem0,
                             add=True)
            wait_scatter(rows1_v, ssem1)
            wait_scatter(rows0_v, ssem0)

    run_phase(0, JP)
    run_phase(JP, J - JP)
    plsc.subcore_barrier()

    @pl.when(s < NS - 1)
    def _():
        pltpu.sync_copy(acc_sh.at[pl.ds(s * SB, SB)],
                        acc_hbm.at[c, pl.ds(s * SB, SB)])

    @pl.when(s == NS - 1)
    def _():
        pltpu.sync_copy(acc_sh.at[pl.ds((NS - 1) * SB, SBL)],
                        acc_hbm.at[c, pl.ds((NS - 1) * SB, SBL)])


def _dis(degp_ref):
    return lax.rsqrt(degp_ref[0, :, 0:1] + degp_ref[1, :, 0:1] + 1.0)


def _dense_in_body(degp_ref, x_ref, w_ref, b_ref, hs_ref):
    hh = jnp.dot(x_ref[...], w_ref[...],
                 preferred_element_type=jnp.float32) + b_ref[...]
    hs_ref[...] = hh * _dis(degp_ref)


def _dense_mid_body(acc_ref, degp_ref, g_ref, be_ref, w_ref, b_ref,
                    out_ref):
    dis = _dis(degp_ref)
    z = (acc_ref[0] + acc_ref[1]) * dis
    mu = jnp.mean(z, axis=0, keepdims=True)
    var = jnp.mean((z - mu) ** 2, axis=0, keepdims=True)
    h = (z - mu) * lax.rsqrt(var + 1e-5) * g_ref[...] + be_ref[...]
    h = jnp.maximum(h, 0.0)
    hh = jnp.dot(h, w_ref[...],
                 preferred_element_type=jnp.float32) + b_ref[...]
    out_ref[...] = hh * dis


def _dense_out_body(acc_ref, degp_ref, g_ref, be_ref, batch_ref,
                    wc1_ref, bc1_ref, wc2_ref, bc2_ref, out_ref):
    z = (acc_ref[0] + acc_ref[1]) * _dis(degp_ref)
    mu = jnp.mean(z, axis=0, keepdims=True)
    var = jnp.mean((z - mu) ** 2, axis=0, keepdims=True)
    h = (z - mu) * lax.rsqrt(var + 1e-5) * g_ref[...] + be_ref[...]
    h = jnp.maximum(h, 0.0)
    gi = lax.broadcasted_iota(jnp.int32, (NG, N), 0)
    onehot = (gi == batch_ref[...]).astype(jnp.float32)
    pool = jnp.dot(onehot, h, preferred_element_type=jnp.float32)
    c1 = jnp.maximum(
        jnp.dot(pool, wc1_ref[...], preferred_element_type=jnp.float32)
        + bc1_ref[...], 0.0)
    out_ref[...] = jnp.dot(c1, wc2_ref[...],
                           preferred_element_type=jnp.float32) + bc2_ref[...]


def kernel(x, edge_index, batch, W1, b1, g1, be1, W2, b2, g2, be2,
           W3, b3, g3, be3, Wc1, bc1, Wc2, bc2):
    f32 = jnp.float32
    i32 = jnp.int32
    ei = edge_index.astype(i32)
    edge_r = ei.reshape(2, NW, J, K)
    edge_d = ei.reshape(2, NW, JD, KD)
    zeros_deg = jnp.zeros((640, DW), f32)
    zeros_rows = jnp.zeros((SB, H), f32)
    batch2d = batch.astype(i32).reshape(1, N)

    degp = _deg_kernel(edge_d, zeros_deg)

    dense_in = pl.pallas_call(
        _dense_in_body,
        out_shape=jax.ShapeDtypeStruct((N, H), f32))
    hs = dense_in(degp, x, W1, b1.reshape(1, H))

    dense_mid = pl.pallas_call(
        _dense_mid_body,
        out_shape=jax.ShapeDtypeStruct((N, H), f32))
    for (g, be, W, b) in ((g1, be1, W2, b2), (g2, be2, W3, b3)):
        acc = _msg_kernel(hs, edge_r, zeros_rows)
        hs = dense_mid(acc, degp, g.reshape(1, H), be.reshape(1, H),
                       W, b.reshape(1, H))

    acc = _msg_kernel(hs, edge_r, zeros_rows)
    dense_out = pl.pallas_call(
        _dense_out_body,
        out_shape=jax.ShapeDtypeStruct((NG, C), f32))
    out = dense_out(acc, degp, g3.reshape(1, H), be3.reshape(1, H),
                    batch2d, Wc1, bc1.reshape(1, H), Wc2, bc2.reshape(1, C))
    return out
